# baseline (speedup 1.0000x reference)
import functools

import jax
import jax.numpy as jnp
from jax import lax
from jax.experimental import pallas as pl
from jax.experimental.pallas import tpu as pltpu
from jax.experimental.pallas import tpu_sc as plsc

NN, EE, FI, NH1, CH, FO, NB = 10000, 320000, 128, 8, 64, 128, 64
NPAD = 10240
NCORES, NSUB = 2, 16
ECH = EE // NSUB
EBUF = 2000
NSUBCH = ECH // EBUF
CMAX = ECH + 80
BM = 256
NRB = NPAD // BM


def _tc1_body(x_ref, w_ref, hp_ref):
    hp_ref[...] = jnp.dot(x_ref[...], w_ref[...],
                          preferred_element_type=jnp.float32)


def _tc1(xp, w1e):
    return pl.pallas_call(
        _tc1_body,
        grid=(NRB, 5),
        in_specs=[pl.BlockSpec((BM, FI), lambda i, k: (i, 0)),
                  pl.BlockSpec((FI, 128), lambda i, k: (0, k))],
        out_specs=pl.BlockSpec((BM, 128), lambda i, k: (k * NRB + i, 0)),
        out_shape=jax.ShapeDtypeStruct((5 * NPAD, 128), jnp.float32),
    )(xp, w1e)


def _tc2_body(b0, b1_, b2_, b3_, b4_, e16_ref, bias_ref, w2g_ref, gp_ref):
    dn16 = b4_[...][:, 0:16]
    dbc = jnp.dot(dn16, e16_ref[...], preferred_element_type=jnp.float32)
    un = jnp.concatenate([b0[...], b1_[...], b2_[...], b3_[...]], axis=1)
    h1 = un / (dbc + 1e-16) + bias_ref[...]
    h1 = jnp.where(h1 > 0.0, h1, jnp.exp(h1) - 1.0)
    gp_ref[...] = jnp.dot(h1, w2g_ref[...], preferred_element_type=jnp.float32)


def _tc2(outp1, e16, b1r, w2g):
    def blk(k):
        return pl.BlockSpec((BM, 128), lambda i, _k=k: (_k * NRB + i, 0))
    return pl.pallas_call(
        _tc2_body,
        grid=(NRB,),
        in_specs=[blk(0), blk(1), blk(2), blk(3), blk(4),
                  pl.BlockSpec((16, 512), lambda i: (0, 0)),
                  pl.BlockSpec((1, 512), lambda i: (0, 0)),
                  pl.BlockSpec((512, 128), lambda i: (0, 0))],
        out_specs=pl.BlockSpec((BM, 128), lambda i: (i, 0)),
        out_shape=jax.ShapeDtypeStruct((NPAD, 128), jnp.float32),
    )(outp1, outp1, outp1, outp1, outp1, e16, b1r, w2g)


def _tc3_body(o_ref, b_ref, e1_ref, b2_ref, wl_ref, bl_ref, out_ref):
    o = o_ref[...]
    un = o[:, :64]
    d64 = jnp.dot(o[:, 64:80], e1_ref[...], preferred_element_type=jnp.float32)
    h2 = un / (d64 + 1e-16) + b2_ref[...]
    h2 = jnp.where(h2 > 0.0, h2, jnp.exp(h2) - 1.0)
    bb = b_ref[...]
    oh = (bb == lax.broadcasted_iota(jnp.int32, (1, NB), 1)).astype(jnp.float32)
    sums = lax.dot_general(oh, h2, (((0,), (0,)), ((), ())),
                           preferred_element_type=jnp.float32)
    ones = jnp.ones((NPAD, 8), jnp.float32)
    cnts = lax.dot_general(oh, ones, (((0,), (0,)), ((), ())),
                           preferred_element_type=jnp.float32)[:, 0:1]
    pooled = sums / jnp.maximum(cnts, 1.0)
    out_ref[...] = jnp.dot(pooled, wl_ref[...],
                           preferred_element_type=jnp.float32) + bl_ref[...]


def _tc3(outp2, batchp, e1, b2r, wl, blr):
    return pl.pallas_call(
        _tc3_body,
        out_shape=jax.ShapeDtypeStruct((NB, FO), jnp.float32),
    )(outp2, batchp, e1, b2r, wl, blr)


def _dg(x, idx):
    return lax.gather(
        x, idx[:, None],
        lax.GatherDimensionNumbers(offset_dims=(), collapsed_slice_dims=(0,),
                                   start_index_map=(0,)),
        (1,), mode=lax.GatherScatterMode.PROMISE_IN_BOUNDS)


def _sc_agg(nblk, ab, att_off, nheads, npass, table, srcarr, dstarr):
    slab_n = NPAD // (NCORES * npass)
    slabr = slab_n + 16
    dummy = slab_n
    rpt = slab_n // NSUB
    nsec = nblk + 1
    mesh = plsc.VectorSubcoreMesh(core_axis_name="c", subcore_axis_name="s",
                                  num_cores=NCORES, num_subcores=NSUB)

    @functools.partial(
        pl.kernel,
        out_type=jax.ShapeDtypeStruct((nblk * NPAD, 128), jnp.float32),
        mesh=mesh,
        scratch_types=[
            pltpu.VMEM((EBUF,), jnp.int32),
            pltpu.VMEM((EBUF,), jnp.int32),
            pltpu.VMEM((CMAX,), jnp.int32),
            pltpu.VMEM((4, nsec * 16), jnp.int32),
            pltpu.VMEM((4, nsec * 16, 128), jnp.float32),
            pltpu.VMEM_SHARED((nblk * slabr, 128), jnp.float32),
            pltpu.SemaphoreType.DMA,
            pltpu.SemaphoreType.DMA,
        ],
    )
    def body(tab_hbm, src_hbm, dst_hbm, out_hbm,
             srcb, dstb, cpk, gidx, rows, slab, sem, sem2):
        cidx = lax.axis_index("c")
        sidx = lax.axis_index("s")
        e0 = sidx * ECH
        iota = lax.broadcasted_iota(jnp.int32, (16,), 0)
        zer = jnp.zeros((16,), jnp.float32)

        def one_pass(p, _):
            base = (cidx * npass + p) * slab_n

            def zrow(j, _):
                for q in range(8):
                    rows[0, j, pl.ds(q * 16, 16)] = zer
                return 0
            lax.fori_loop(0, 16, zrow, 0)

            def zslab(i, _):
                for k in range(nblk):
                    pltpu.sync_copy(
                        rows.at[0, pl.ds(0, 16)],
                        slab.at[pl.ds(k * slabr + sidx * rpt + i * 16, 16)])
                return 0
            lax.fori_loop(0, rpt // 16, zslab, 0)
            plsc.subcore_barrier()

            def fsub(ss, carry):
                off = e0 + ss * EBUF
                pltpu.sync_copy(src_hbm.at[pl.ds(off, EBUF)], srcb)
                pltpu.sync_copy(dst_hbm.at[pl.ds(off, EBUF)], dstb)

                def fin(i, carry2):
                    c16, pc, ppk = carry2
                    dv = dstb[pl.ds(i * 16, 16)]
                    sv = srcb[pl.ds(i * 16, 16)]
                    lv = dv - base
                    m = (lv >= 0) & (lv < slab_n)
                    pk = sv | (lv << 14)
                    r = jnp.where(m, 1, 0)
                    for k in (1, 2, 4, 8):
                        sh = _dg(r, jnp.maximum(iota - k, 0))
                        r = r + jnp.where(iota >= k, sh, 0)
                    tot = r[15]
                    lo = jnp.zeros((16,), jnp.int32)
                    hi = jnp.full((16,), 16, jnp.int32)
                    for _ in range(5):
                        mid = (lo + hi) >> 1
                        rm = _dg(r, jnp.minimum(mid, 15))
                        th = rm < (iota + 1)
                        lo = jnp.where(th, mid + 1, lo)
                        hi = jnp.where(th, hi, mid)
                    sel = jnp.minimum(lo, 15)
                    cv = _dg(pk, sel)
                    takep = iota < pc
                    ib = jnp.minimum(jnp.maximum(iota - pc, 0), 15)
                    mg = jnp.where(takep, ppk, _dg(cv, ib))
                    total = pc + tot
                    c16a = pl.multiple_of(c16, 16)
                    cpk[pl.ds(c16a, 16)] = mg
                    full = total >= 16
                    isf = jnp.minimum(iota + 16 - pc, 15)
                    nppk = jnp.where(full, _dg(cv, isf), mg)
                    c16n = jnp.where(full, c16 + 16, c16)
                    pcn = jnp.where(full, total - 16, total)
                    return (c16n, pcn, nppk)
                return lax.fori_loop(0, EBUF // 16, fin, carry)
            zi = jnp.zeros((16,), jnp.int32)
            c16f, pcf, ppkf = lax.fori_loop(0, NSUBCH, fsub, (0, 0, zi))
            c16fa = pl.multiple_of(c16f, 16)
            cpk[pl.ds(c16fa, 16)] = ppkf
            count = c16f + pcf
            nbat = (count + 15) >> 4

            def fire_g(b, buf):
                pk = cpk[pl.ds(b * 16, 16)]
                vm = (b * 16 + iota) < count
                sv = jnp.where(vm, pk & 16383, 0)
                dg_ = jnp.where(vm, base + (pk >> 14), 0)
                for k in range(nblk):
                    gidx[buf, pl.ds(k * 16, 16)] = sv + k * NPAD
                gidx[buf, pl.ds(nblk * 16, 16)] = dg_ + ab * NPAD
                pltpu.async_copy(tab_hbm.at[gidx.at[buf]], rows.at[buf], sem)

            def drain_g(buf):
                pltpu.make_async_copy(tab_hbm.at[pl.ds(0, nsec * 16)],
                                      rows.at[buf], sem).wait()

            def compute(b, buf):
                pk = cpk[pl.ds(b * 16, 16)]
                vm = (b * 16 + iota) < count
                lv = pk >> 14
                lvs = jnp.where(vm, lv, dummy)
                for j in range(16):
                    asrc = rows[buf, ab * 16 + j, pl.ds(att_off, 16)]
                    adrow = rows[buf, nblk * 16 + j, pl.ds(att_off + 16, 16)]
                    al = asrc + adrow
                    al = jnp.where(al > 0.0, al, 0.2 * al)
                    exv = jnp.exp(al)
                    rows[buf, ab * 16 + j, pl.ds(att_off, 16)] = exv
                    mlts = {}
                    for k in range(nblk):
                        for q in range(8):
                            gc = k * 128 + q * 16
                            if k == ab and att_off <= gc < att_off + 16:
                                continue
                            h = gc // CH
                            if h >= nheads:
                                continue
                            if h not in mlts:
                                mlts[h] = jnp.full((16,), exv[h], jnp.float32)
                            rows[buf, k * 16 + j, pl.ds(q * 16, 16)] = (
                                rows[buf, k * 16 + j, pl.ds(q * 16, 16)]
                                * mlts[h])
                for k in range(nblk):
                    pltpu.async_copy(rows.at[buf, pl.ds(k * 16, 16)],
                                     slab.at[lvs + k * slabr], sem2, add=True)

            def drain_sc():
                for k in range(nblk):
                    pltpu.make_async_copy(tab_hbm.at[pl.ds(0, 16)],
                                          slab.at[pl.ds(0, 16)], sem2).wait()

            nb4 = (nbat + 3) >> 2
            tot = nb4 * 4

            @pl.when(nbat > 0)
            def _():
                fire_g(0, 0)
                fire_g(1, 1)

            def pb(i, _):
                for s in range(4):
                    b = i * 4 + s

                    if s < 2:
                        @pl.when(b >= 2)
                        def _():
                            drain_sc()
                        fire_g(b + 2, (s + 2) % 4)
                    else:
                        drain_sc()

                        @pl.when(i + 1 < nb4)
                        def _(_b=b, _s=s):
                            fire_g(_b + 2, (_s + 2) % 4)
                    drain_g(s)
                    compute(b, s)
                return 0
            lax.fori_loop(0, nb4, pb, 0)

            @pl.when(nbat > 0)
            def _():
                drain_sc()
                drain_sc()
            plsc.subcore_barrier()
            for k in range(nblk):
                pltpu.sync_copy(
                    slab.at[pl.ds(k * slabr + sidx * rpt, rpt)],
                    out_hbm.at[pl.ds(k * NPAD + base + sidx * rpt, rpt)])
            return 0
        lax.fori_loop(0, npass, one_pass, 0)

    return body(table, srcarr, dstarr)


def kernel(x, edge_index, batch, W1, att_src1, att_dst1, b1,
           W2, att_src2, att_dst2, b2, Wl, bl):
    f32 = jnp.float32
    as1 = att_src1.reshape(NH1 * CH).astype(f32)
    ad1 = att_dst1.reshape(NH1 * CH).astype(f32)
    idx = jnp.arange(NH1 * CH)
    v1s = jnp.zeros((NH1 * CH, 16), f32).at[idx, idx // CH].set(as1)
    v1d = jnp.zeros((NH1 * CH, 16), f32).at[idx, idx // CH].set(ad1)
    w1e = jnp.concatenate([W1, W1 @ v1s, W1 @ v1d,
                           jnp.zeros((FI, 96), f32)], axis=1)
    e16 = jnp.zeros((16, 512), f32).at[idx // CH, idx].set(1.0)
    w2g = (jnp.zeros((512, 128), f32).at[:, 0:64].set(W2)
           .at[:, 64].set(W2 @ att_src2.reshape(CH))
           .at[:, 80].set(W2 @ att_dst2.reshape(CH)))
    e1 = jnp.zeros((16, 64), f32).at[0, :].set(1.0)

    xp = jnp.pad(x, ((0, NPAD - NN), (0, 0)))
    src = edge_index[0]
    dst = edge_index[1]
    batchp = jnp.pad(batch, (0, NPAD - NN), constant_values=NB).reshape(NPAD, 1)

    table1 = _tc1(xp, w1e)
    outp1 = _sc_agg(5, 4, 0, NH1, 4, table1, src, dst)
    gplus = _tc2(outp1, e16, b1.reshape(1, 512), w2g)
    outp2 = _sc_agg(1, 0, 64, 1, 1, gplus, src, dst)
    return _tc3(outp2, batchp, e1, b2.reshape(1, 64), Wl, bl.reshape(1, FO))


# speedup vs baseline: 1.0092x; 1.0092x over previous
import functools

import jax
import jax.numpy as jnp
from jax import lax
from jax.experimental import pallas as pl
from jax.experimental.pallas import tpu as pltpu
from jax.experimental.pallas import tpu_sc as plsc

NN, EE, FI, NH1, CH, FO, NB = 10000, 320000, 128, 8, 64, 128, 64
NPAD = 10240
NCORES, NSUB = 2, 16
SLAB = 1280
NPASS = 4
SLABR = SLAB + 16
DUMMY = SLAB
ECH = EE // NSUB
EBUF = 2000
NSUBCH = ECH // EBUF
CMAX = ECH + 80
RPT = SLAB // NSUB
BM = 256
NRB = NPAD // BM


def _tc1_body(x_ref, w_ref, wd_ref, hp_ref, ad_ref):
    xb = x_ref[...]
    hp_ref[...] = jnp.dot(xb, w_ref[...], preferred_element_type=jnp.float32)
    ad_ref[...] = jnp.dot(xb, wd_ref[...], preferred_element_type=jnp.float32)


def _tc1(xp, w1e, wd):
    return pl.pallas_call(
        _tc1_body,
        grid=(NRB, 5),
        in_specs=[pl.BlockSpec((BM, FI), lambda i, k: (i, 0)),
                  pl.BlockSpec((FI, 128), lambda i, k: (0, k)),
                  pl.BlockSpec((FI, 16), lambda i, k: (0, 0))],
        out_specs=[pl.BlockSpec((BM, 128), lambda i, k: (k * NRB + i, 0)),
                   pl.BlockSpec((BM, 16), lambda i, k: (i, 0))],
        out_shape=[jax.ShapeDtypeStruct((5 * NPAD, 128), jnp.float32),
                   jax.ShapeDtypeStruct((NPAD, 16), jnp.float32)],
    )(xp, w1e, wd)


def _tc2_body(b0, b1_, b2_, b3_, b4_, e16_ref, bias_ref, w2g_ref, w2d_ref,
              gp_ref, bd_ref):
    dn16 = b4_[...][:, 0:16]
    dbc = jnp.dot(dn16, e16_ref[...], preferred_element_type=jnp.float32)
    un = jnp.concatenate([b0[...], b1_[...], b2_[...], b3_[...]], axis=1)
    h1 = un / (dbc + 1e-16) + bias_ref[...]
    h1 = jnp.where(h1 > 0.0, h1, jnp.exp(h1) - 1.0)
    gp_ref[...] = jnp.dot(h1, w2g_ref[...], preferred_element_type=jnp.float32)
    bd_ref[...] = jnp.dot(h1, w2d_ref[...], preferred_element_type=jnp.float32)


def _tc2(outp1, e16, b1r, w2g, w2d):
    def blk(k):
        return pl.BlockSpec((BM, 128), lambda i, _k=k: (_k * NRB + i, 0))
    return pl.pallas_call(
        _tc2_body,
        grid=(NRB,),
        in_specs=[blk(0), blk(1), blk(2), blk(3), blk(4),
                  pl.BlockSpec((16, 512), lambda i: (0, 0)),
                  pl.BlockSpec((1, 512), lambda i: (0, 0)),
                  pl.BlockSpec((512, 128), lambda i: (0, 0)),
                  pl.BlockSpec((512, 16), lambda i: (0, 0))],
        out_specs=[pl.BlockSpec((BM, 128), lambda i: (i, 0)),
                   pl.BlockSpec((BM, 16), lambda i: (i, 0))],
        out_shape=[jax.ShapeDtypeStruct((NPAD, 128), jnp.float32),
                   jax.ShapeDtypeStruct((NPAD, 16), jnp.float32)],
    )(outp1, outp1, outp1, outp1, outp1, e16, b1r, w2g, w2d)


def _tc3_body(o_ref, b_ref, e1_ref, b2_ref, wl_ref, bl_ref, out_ref):
    o = o_ref[...]
    un = o[:, :64]
    d64 = jnp.dot(o[:, 64:80], e1_ref[...], preferred_element_type=jnp.float32)
    h2 = un / (d64 + 1e-16) + b2_ref[...]
    h2 = jnp.where(h2 > 0.0, h2, jnp.exp(h2) - 1.0)
    bb = b_ref[...]
    oh = (bb == lax.broadcasted_iota(jnp.int32, (1, NB), 1)).astype(jnp.float32)
    sums = lax.dot_general(oh, h2, (((0,), (0,)), ((), ())),
                           preferred_element_type=jnp.float32)
    ones = jnp.ones((NPAD, 8), jnp.float32)
    cnts = lax.dot_general(oh, ones, (((0,), (0,)), ((), ())),
                           preferred_element_type=jnp.float32)[:, 0:1]
    pooled = sums / jnp.maximum(cnts, 1.0)
    out_ref[...] = jnp.dot(pooled, wl_ref[...],
                           preferred_element_type=jnp.float32) + bl_ref[...]


def _tc3(outp2, batchp, e1, b2r, wl, blr):
    return pl.pallas_call(
        _tc3_body,
        out_shape=jax.ShapeDtypeStruct((NB, FO), jnp.float32),
    )(outp2, batchp, e1, b2r, wl, blr)


def _dg(x, idx):
    return lax.gather(
        x, idx[:, None],
        lax.GatherDimensionNumbers(offset_dims=(), collapsed_slice_dims=(0,),
                                   start_index_map=(0,)),
        (1,), mode=lax.GatherScatterMode.PROMISE_IN_BOUNDS)


def _sc_agg(nblk, ab, att_off, nheads, npass, adw, table, adtab, srcarr,
            dstarr):
    slab_n = NPAD // (NCORES * npass)
    slabr = slab_n + 16
    dummy = slab_n
    rpt = slab_n // NSUB
    mesh = plsc.VectorSubcoreMesh(core_axis_name="c", subcore_axis_name="s",
                                  num_cores=NCORES, num_subcores=NSUB)

    @functools.partial(
        pl.kernel,
        out_type=jax.ShapeDtypeStruct((nblk * NPAD, 128), jnp.float32),
        mesh=mesh,
        scratch_types=[
            pltpu.VMEM((EBUF,), jnp.int32),
            pltpu.VMEM((EBUF,), jnp.int32),
            pltpu.VMEM((CMAX,), jnp.int32),
            pltpu.VMEM((slab_n * adw + 16,), jnp.float32),
            pltpu.VMEM((4, nblk * 16), jnp.int32),
            pltpu.VMEM((4, nblk * 16, 128), jnp.float32),
            pltpu.VMEM_SHARED((nblk * slabr, 128), jnp.float32),
            pltpu.SemaphoreType.DMA,
            pltpu.SemaphoreType.DMA,
        ],
    )
    def body(tab_hbm, ad_hbm, src_hbm, dst_hbm, out_hbm,
             srcb, dstb, cpk, adl, gidx, rows, slab, sem, sem2):
        cidx = lax.axis_index("c")
        sidx = lax.axis_index("s")
        e0 = sidx * ECH
        iota = lax.broadcasted_iota(jnp.int32, (16,), 0)
        zer = jnp.zeros((16,), jnp.float32)

        def one_pass(p, _):
            base = (cidx * npass + p) * slab_n
            pltpu.sync_copy(ad_hbm.at[pl.ds(base * adw, slab_n * adw)],
                            adl.at[pl.ds(0, slab_n * adw)])

            def zrow(j, _):
                for q in range(8):
                    rows[0, j, pl.ds(q * 16, 16)] = zer
                return 0
            lax.fori_loop(0, 16, zrow, 0)

            def zslab(i, _):
                for k in range(nblk):
                    pltpu.sync_copy(
                        rows.at[0, pl.ds(0, 16)],
                        slab.at[pl.ds(k * slabr + sidx * rpt + i * 16, 16)])
                return 0
            lax.fori_loop(0, rpt // 16, zslab, 0)
            plsc.subcore_barrier()

            def fsub(ss, carry):
                off = e0 + ss * EBUF
                pltpu.sync_copy(src_hbm.at[pl.ds(off, EBUF)], srcb)
                pltpu.sync_copy(dst_hbm.at[pl.ds(off, EBUF)], dstb)

                def fin(i, carry2):
                    c16, pc, ppk = carry2
                    dv = dstb[pl.ds(i * 16, 16)]
                    sv = srcb[pl.ds(i * 16, 16)]
                    lv = dv - base
                    m = (lv >= 0) & (lv < slab_n)
                    pk = sv | (lv << 14)
                    r = jnp.where(m, 1, 0)
                    for k in (1, 2, 4, 8):
                        sh = _dg(r, jnp.maximum(iota - k, 0))
                        r = r + jnp.where(iota >= k, sh, 0)
                    tot = r[15]
                    lo = jnp.zeros((16,), jnp.int32)
                    hi = jnp.full((16,), 16, jnp.int32)
                    for _ in range(5):
                        mid = (lo + hi) >> 1
                        rm = _dg(r, jnp.minimum(mid, 15))
                        th = rm < (iota + 1)
                        lo = jnp.where(th, mid + 1, lo)
                        hi = jnp.where(th, hi, mid)
                    sel = jnp.minimum(lo, 15)
                    cv = _dg(pk, sel)
                    takep = iota < pc
                    ib = jnp.minimum(jnp.maximum(iota - pc, 0), 15)
                    mg = jnp.where(takep, ppk, _dg(cv, ib))
                    total = pc + tot
                    c16a = pl.multiple_of(c16, 16)
                    cpk[pl.ds(c16a, 16)] = mg
                    full = total >= 16
                    isf = jnp.minimum(iota + 16 - pc, 15)
                    nppk = jnp.where(full, _dg(cv, isf), mg)
                    c16n = jnp.where(full, c16 + 16, c16)
                    pcn = jnp.where(full, total - 16, total)
                    return (c16n, pcn, nppk)
                return lax.fori_loop(0, EBUF // 16, fin, carry)
            zi = jnp.zeros((16,), jnp.int32)
            c16f, pcf, ppkf = lax.fori_loop(0, NSUBCH, fsub, (0, 0, zi))
            c16fa = pl.multiple_of(c16f, 16)
            cpk[pl.ds(c16fa, 16)] = ppkf
            count = c16f + pcf
            nbat = (count + 15) >> 4

            def fire_g(b, buf):
                pk = cpk[pl.ds(b * 16, 16)]
                vm = (b * 16 + iota) < count
                sv = jnp.where(vm, pk & 16383, 0)
                for k in range(nblk):
                    gidx[buf, pl.ds(k * 16, 16)] = sv + k * NPAD
                pltpu.async_copy(tab_hbm.at[gidx.at[buf]], rows.at[buf], sem)

            def drain_g(buf):
                pltpu.make_async_copy(tab_hbm.at[pl.ds(0, nblk * 16)],
                                      rows.at[buf], sem).wait()

            def compute(b, buf):
                pk = cpk[pl.ds(b * 16, 16)]
                vm = (b * 16 + iota) < count
                lv = pk >> 14
                lva = jnp.where(vm, lv, 0)
                lvs = jnp.where(vm, lv, dummy)
                for j in range(16):
                    asrc = rows[buf, ab * 16 + j, pl.ds(att_off, 16)]
                    if adw == 8:
                        adrow = adl[pl.ds(lva[j] * 8, 16)]
                    else:
                        r0 = lva[j]
                        rb = (r0 >> 3) << 3
                        av = adl[pl.ds(rb, 16)]
                        adrow = _dg(av, jnp.full((16,), r0 & 7, jnp.int32))
                    al = asrc + adrow
                    al = jnp.where(al > 0.0, al, 0.2 * al)
                    exv = jnp.exp(al)
                    rows[buf, ab * 16 + j, pl.ds(att_off, 16)] = exv
                    mlts = {}
                    for k in range(nblk):
                        for q in range(8):
                            gc = k * 128 + q * 16
                            if k == ab and att_off <= gc < att_off + 16:
                                continue
                            h = gc // CH
                            if h >= nheads:
                                continue
                            if h not in mlts:
                                mlts[h] = jnp.full((16,), exv[h], jnp.float32)
                            rows[buf, k * 16 + j, pl.ds(q * 16, 16)] = (
                                rows[buf, k * 16 + j, pl.ds(q * 16, 16)]
                                * mlts[h])
                for k in range(nblk):
                    pltpu.async_copy(rows.at[buf, pl.ds(k * 16, 16)],
                                     slab.at[lvs + k * slabr], sem2, add=True)

            def drain_sc():
                for k in range(nblk):
                    pltpu.make_async_copy(tab_hbm.at[pl.ds(0, 16)],
                                          slab.at[pl.ds(0, 16)], sem2).wait()

            nb4 = (nbat + 3) >> 2
            tot = nb4 * 4

            @pl.when(nbat > 0)
            def _():
                fire_g(0, 0)
                fire_g(1, 1)

            def pb(i, _):
                for s in range(4):
                    b = i * 4 + s

                    if s < 2:
                        @pl.when(b >= 2)
                        def _():
                            drain_sc()
                        fire_g(b + 2, (s + 2) % 4)
                    else:
                        drain_sc()

                        @pl.when(i + 1 < nb4)
                        def _(_b=b, _s=s):
                            fire_g(_b + 2, (_s + 2) % 4)
                    drain_g(s)
                    compute(b, s)
                return 0
            lax.fori_loop(0, nb4, pb, 0)

            @pl.when(nbat > 0)
            def _():
                drain_sc()
                drain_sc()
            plsc.subcore_barrier()
            for k in range(nblk):
                pltpu.sync_copy(
                    slab.at[pl.ds(k * slabr + sidx * rpt, rpt)],
                    out_hbm.at[pl.ds(k * NPAD + base + sidx * rpt, rpt)])
            return 0
        lax.fori_loop(0, npass, one_pass, 0)

    return body(table, adtab, srcarr, dstarr)


def kernel(x, edge_index, batch, W1, att_src1, att_dst1, b1,
           W2, att_src2, att_dst2, b2, Wl, bl):
    f32 = jnp.float32
    as1 = att_src1.reshape(NH1 * CH).astype(f32)
    ad1 = att_dst1.reshape(NH1 * CH).astype(f32)
    idx = jnp.arange(NH1 * CH)
    v1s = jnp.zeros((NH1 * CH, 16), f32).at[idx, idx // CH].set(as1)
    v1d = jnp.zeros((NH1 * CH, 16), f32).at[idx, idx // CH].set(ad1)
    w1e = jnp.concatenate([W1, W1 @ v1s, jnp.zeros((FI, 112), f32)], axis=1)
    wd = W1 @ v1d
    e16 = jnp.zeros((16, 512), f32).at[idx // CH, idx].set(1.0)
    w2g = (jnp.zeros((512, 128), f32).at[:, 0:64].set(W2)
           .at[:, 64].set(W2 @ att_src2.reshape(CH)))
    w2d = jnp.zeros((512, 16), f32).at[:, 0].set(W2 @ att_dst2.reshape(CH))
    e1 = jnp.zeros((16, 64), f32).at[0, :].set(1.0)

    xp = jnp.pad(x, ((0, NPAD - NN), (0, 0)))
    src = edge_index[0]
    dst = edge_index[1]
    batchp = jnp.pad(batch, (0, NPAD - NN), constant_values=NB).reshape(NPAD, 1)

    table1, adst = _tc1(xp, w1e, wd)
    outp1 = _sc_agg(5, 4, 0, NH1, 4, 8, table1,
                    adst[:, 0:8].reshape(NPAD * 8), src, dst)
    gplus, bdst2 = _tc2(outp1, e16, b1.reshape(1, 512), w2g, w2d)
    outp2 = _sc_agg(1, 0, 64, 1, 1, 1, gplus, bdst2[:, 0].reshape(NPAD),
                    src, dst)
    return _tc3(outp2, batchp, e1, b2.reshape(1, 64), Wl, bl.reshape(1, FO))
